# Initial kernel scaffold; baseline (speedup 1.0000x reference)
import jax
import jax.numpy as jnp
from jax.experimental import pallas as pl


def kernel(v_high_feat, coarse_attn_map):
    raise NotImplementedError("write your pallas kernel here")


# speedup vs baseline: 186.5184x; 186.5184x over previous
import functools

import jax
import jax.numpy as jnp
from jax import lax
from jax.experimental import pallas as pl
from jax.experimental.pallas import tpu as pltpu
from jax.experimental.pallas import tpu_sc as plsc

RATIO = 4
C = 96
H = 224
W = 224
HL = H // RATIO
WL = W // RATIO
NL = HL * WL

NC = 2
NS = 16
NW = NC * NS
BPW = 104
NLP = NW * BPW


def _pool_matrices():
    hh = lax.broadcasted_iota(jnp.int32, (H, HL), 0)
    rr = lax.broadcasted_iota(jnp.int32, (H, HL), 1)
    a = (hh // RATIO == rr).astype(jnp.float32)
    tr = lax.broadcasted_iota(jnp.int32, (HL, H), 0)
    th = lax.broadcasted_iota(jnp.int32, (HL, H), 1)
    at = (th // RATIO == tr).astype(jnp.float32)
    return a, at


def _argmax_body(x_ref, o_ref):
    x = x_ref[...]
    mx = jnp.max(x, axis=1, keepdims=True)
    ii = lax.broadcasted_iota(jnp.int32, x.shape, 1)
    cand = jnp.where(x == mx, ii, NL)
    o_ref[...] = jnp.min(cand, axis=1, keepdims=True)


def _pool_body(x_ref, o_ref):
    a, at = _pool_matrices()
    cb = x_ref.shape[0]
    for c in range(cb):
        x = x_ref[c]
        u = lax.dot(x, a, precision=lax.Precision.HIGHEST)
        z = lax.dot(at, u, precision=lax.Precision.HIGHEST)
        o_ref[c] = z * (1.0 / (RATIO * RATIO))


def _upsample_body(g_ref, o_ref):
    a, at = _pool_matrices()
    cb = g_ref.shape[0]
    for c in range(cb):
        g = g_ref[c]
        r1 = lax.dot(a, g, precision=lax.Precision.HIGHEST)
        o_ref[c] = lax.dot(r1, at, precision=lax.Precision.HIGHEST)


def _sc_gather_body(table_hbm, idx_hbm, out_hbm, idx_v, rows_v, sem):
    wid = lax.axis_index("s") * NC + lax.axis_index("c")
    base = wid * BPW
    pltpu.sync_copy(idx_hbm.at[pl.ds(base, BPW)], idx_v)
    pltpu.async_copy(table_hbm.at[idx_v], rows_v, sem).wait()
    pltpu.sync_copy(rows_v, out_hbm.at[pl.ds(base, BPW)])


def _sc_gather(table, idx_padded):
    mesh = plsc.VectorSubcoreMesh(core_axis_name="c", subcore_axis_name="s")
    return pl.kernel(
        _sc_gather_body,
        out_type=jax.ShapeDtypeStruct((NLP, C), jnp.float32),
        mesh=mesh,
        scratch_types=[
            pltpu.VMEM((BPW,), jnp.int32),
            pltpu.VMEM((BPW, C), jnp.float32),
            pltpu.SemaphoreType.DMA,
        ],
        compiler_params=pltpu.CompilerParams(use_tc_tiling_on_sc=False),
    )(table, idx_padded)


RB = 392
CB = 12


def kernel(v_high_feat, coarse_attn_map):
    attn = coarse_attn_map.reshape(NL, NL)
    x = v_high_feat.reshape(C, H, W)

    idx2 = pl.pallas_call(
        _argmax_body,
        grid=(NL // RB,),
        in_specs=[pl.BlockSpec((RB, NL), lambda i: (i, 0))],
        out_specs=pl.BlockSpec((RB, 1), lambda i: (i, 0)),
        out_shape=jax.ShapeDtypeStruct((NL, 1), jnp.int32),
    )(attn)

    pooled = pl.pallas_call(
        _pool_body,
        grid=(C // CB,),
        in_specs=[pl.BlockSpec((CB, H, W), lambda i: (i, 0, 0))],
        out_specs=pl.BlockSpec((CB, HL, WL), lambda i: (i, 0, 0)),
        out_shape=jax.ShapeDtypeStruct((C, HL, WL), jnp.float32),
    )(x)

    table = pooled.reshape(C, NL).T
    idx_padded = jnp.concatenate(
        [idx2[:, 0], jnp.zeros((NLP - NL,), jnp.int32)])

    gathered = _sc_gather(table, idx_padded)

    gt = gathered[:NL].T.reshape(C, HL, WL)

    out = pl.pallas_call(
        _upsample_body,
        grid=(C // CB,),
        in_specs=[pl.BlockSpec((CB, HL, WL), lambda i: (i, 0, 0))],
        out_specs=pl.BlockSpec((CB, H, W), lambda i: (i, 0, 0)),
        out_shape=jax.ShapeDtypeStruct((C, H, W), jnp.float32),
    )(gt)

    return out.reshape(1, C, H, W)


# speedup vs baseline: 232.5233x; 1.2467x over previous
import functools

import jax
import jax.numpy as jnp
from jax import lax
from jax.experimental import pallas as pl
from jax.experimental.pallas import tpu as pltpu
from jax.experimental.pallas import tpu_sc as plsc

RATIO = 4
C = 96
H = 224
W = 224
HL = H // RATIO
WL = W // RATIO
NL = HL * WL

NC = 2
NS = 16
NW = NC * NS
CPW = C // NW
NVREG = NL // 16

RB = 392
CB = 12


def _pool_matrices():
    hh = lax.broadcasted_iota(jnp.int32, (H, HL), 0)
    rr = lax.broadcasted_iota(jnp.int32, (H, HL), 1)
    a = (hh // RATIO == rr).astype(jnp.float32)
    tr = lax.broadcasted_iota(jnp.int32, (HL, H), 0)
    th = lax.broadcasted_iota(jnp.int32, (HL, H), 1)
    at = (th // RATIO == tr).astype(jnp.float32)
    return a, at


def _argmax_pool_body(attn_ref, x_ref, idx_ref, pool_ref):
    x = attn_ref[...]
    mx = jnp.max(x, axis=1, keepdims=True)
    ii = lax.broadcasted_iota(jnp.int32, x.shape, 1)
    cand = jnp.where(x == mx, ii, NL)
    idx_ref[...] = jnp.min(cand, axis=1, keepdims=True)

    a, _ = _pool_matrices()
    y = (x_ref[:, :, 0, :] + x_ref[:, :, 1, :]
         + x_ref[:, :, 2, :] + x_ref[:, :, 3, :])
    y2 = y.reshape(y.shape[0] * HL, W)
    z = lax.dot(y2, a, precision=lax.Precision.HIGHEST)
    pool_ref[...] = z * (1.0 / (RATIO * RATIO))


def _upsample_body(g_ref, o_ref):
    _, at = _pool_matrices()
    g = g_ref[...]
    z = lax.dot(g, at, precision=lax.Precision.HIGHEST)
    z3 = z.reshape(z.shape[0] // HL, HL, W)
    for r in range(RATIO):
        o_ref[:, :, r, :] = z3


def _sc_gather_body(pcm_hbm, idx_hbm, out_hbm, tbl_v, idx_v, out_v):
    wid = lax.axis_index("s") * NC + lax.axis_index("c")
    base_c = wid * CPW
    pltpu.sync_copy(pcm_hbm.at[pl.ds(base_c, CPW)], tbl_v)
    pltpu.sync_copy(idx_hbm, idx_v)

    def body(g, carry):
        idx16 = idx_v[pl.ds(g * 16, 16)]
        for c in range(CPW):
            cvec = jnp.full((16,), c, jnp.int32)
            vals = plsc.load_gather(tbl_v, [cvec, idx16])
            out_v[c, pl.ds(g * 16, 16)] = vals
        return carry

    lax.fori_loop(0, NVREG, body, 0, unroll=4)
    pltpu.sync_copy(out_v, out_hbm.at[pl.ds(base_c, CPW)])


def _sc_gather(pcm, idx):
    mesh = plsc.VectorSubcoreMesh(core_axis_name="c", subcore_axis_name="s")
    return pl.kernel(
        _sc_gather_body,
        out_type=jax.ShapeDtypeStruct((C, NL), jnp.float32),
        mesh=mesh,
        scratch_types=[
            pltpu.VMEM((CPW, NL), jnp.float32),
            pltpu.VMEM((NL,), jnp.int32),
            pltpu.VMEM((CPW, NL), jnp.float32),
        ],
        compiler_params=pltpu.CompilerParams(
            use_tc_tiling_on_sc=False, needs_layout_passes=False),
    )(pcm, idx)


def kernel(v_high_feat, coarse_attn_map):
    attn = coarse_attn_map.reshape(NL, NL)
    x = v_high_feat.reshape(C, HL, RATIO, W)

    idx2, pooled = pl.pallas_call(
        _argmax_pool_body,
        grid=(NL // RB,),
        in_specs=[
            pl.BlockSpec((RB, NL), lambda i: (i, 0)),
            pl.BlockSpec((CB, HL, RATIO, W), lambda i: (i, 0, 0, 0)),
        ],
        out_specs=[
            pl.BlockSpec((RB, 1), lambda i: (i, 0)),
            pl.BlockSpec((CB * HL, WL), lambda i: (i, 0)),
        ],
        out_shape=[
            jax.ShapeDtypeStruct((NL, 1), jnp.int32),
            jax.ShapeDtypeStruct((C * HL, WL), jnp.float32),
        ],
    )(attn, x)

    gt = _sc_gather(pooled.reshape(C, NL), idx2.reshape(NL))

    out = pl.pallas_call(
        _upsample_body,
        grid=(C // CB,),
        in_specs=[pl.BlockSpec((CB * HL, WL), lambda i: (i, 0))],
        out_specs=pl.BlockSpec((CB, HL, RATIO, W), lambda i: (i, 0, 0, 0)),
        out_shape=jax.ShapeDtypeStruct((C, HL, RATIO, W), jnp.float32),
    )(gt.reshape(C * HL, WL))

    return out.reshape(1, C, H, W)


# speedup vs baseline: 385.5896x; 1.6583x over previous
import functools

import jax
import jax.numpy as jnp
from jax import lax
from jax.experimental import pallas as pl
from jax.experimental.pallas import tpu as pltpu
from jax.experimental.pallas import tpu_sc as plsc

RATIO = 4
C = 96
H = 224
W = 224
HL = H // RATIO
WL = W // RATIO
NL = HL * WL

NC = 2
NS = 16
NW = NC * NS
CPW = C // NW
NVREG = NL // 16

RB = 392
CB = 12

_PREC = lax.Precision.DEFAULT


def _pool_matrices():
    hh = lax.broadcasted_iota(jnp.int32, (H, HL), 0)
    rr = lax.broadcasted_iota(jnp.int32, (H, HL), 1)
    a = (hh // RATIO == rr).astype(jnp.float32)
    tr = lax.broadcasted_iota(jnp.int32, (HL, H), 0)
    th = lax.broadcasted_iota(jnp.int32, (HL, H), 1)
    at = (th // RATIO == tr).astype(jnp.float32)
    return a, at


def _argmax_body(x_ref, o_ref):
    x = x_ref[...]
    mx = jnp.max(x, axis=1, keepdims=True)
    ii = lax.broadcasted_iota(jnp.int32, x.shape, 1)
    cand = jnp.where(x == mx, ii, NL)
    o_ref[...] = jnp.min(cand, axis=1, keepdims=True)


def _pool_body(x_ref, o_ref):
    a, at = _pool_matrices()
    x = x_ref[...]
    u = lax.dot(x.reshape(CB * H, W), a, precision=_PREC)
    zs = [lax.dot(at, u[c * H:(c + 1) * H], precision=_PREC)
          for c in range(CB)]
    z = jnp.concatenate(zs, axis=0)
    o_ref[...] = z * (1.0 / (RATIO * RATIO))


def _upsample_body(g_ref, o_ref):
    a, at = _pool_matrices()
    g = g_ref[...]
    rs = [lax.dot(a, g[c * HL:(c + 1) * HL], precision=_PREC)
          for c in range(CB)]
    r = jnp.concatenate(rs, axis=0)
    z = lax.dot(r, at, precision=_PREC)
    o_ref[...] = z.reshape(CB, H, W)


def _sc_gather_body(pcm_hbm, idx_hbm, out_hbm, tbl_v, idx_v, out_v):
    wid = lax.axis_index("s") * NC + lax.axis_index("c")
    base_c = wid * CPW
    pltpu.sync_copy(pcm_hbm.at[pl.ds(base_c, CPW)], tbl_v)
    pltpu.sync_copy(idx_hbm, idx_v)

    def body(g, carry):
        idx16 = idx_v[pl.ds(g * 16, 16)]
        for c in range(CPW):
            cvec = jnp.full((16,), c, jnp.int32)
            vals = plsc.load_gather(tbl_v, [cvec, idx16])
            out_v[c, pl.ds(g * 16, 16)] = vals
        return carry

    lax.fori_loop(0, NVREG, body, 0, unroll=4)
    pltpu.sync_copy(out_v, out_hbm.at[pl.ds(base_c, CPW)])


def _sc_gather(pcm, idx):
    mesh = plsc.VectorSubcoreMesh(core_axis_name="c", subcore_axis_name="s")
    return pl.kernel(
        _sc_gather_body,
        out_type=jax.ShapeDtypeStruct((C, NL), jnp.float32),
        mesh=mesh,
        scratch_types=[
            pltpu.VMEM((CPW, NL), jnp.float32),
            pltpu.VMEM((NL,), jnp.int32),
            pltpu.VMEM((CPW, NL), jnp.float32),
        ],
        compiler_params=pltpu.CompilerParams(
            use_tc_tiling_on_sc=False, needs_layout_passes=False),
    )(pcm, idx)


def kernel(v_high_feat, coarse_attn_map):
    attn = coarse_attn_map.reshape(NL, NL)
    x = v_high_feat.reshape(C, H, W)

    idx2 = pl.pallas_call(
        _argmax_body,
        grid=(NL // RB,),
        in_specs=[pl.BlockSpec((RB, NL), lambda i: (i, 0))],
        out_specs=pl.BlockSpec((RB, 1), lambda i: (i, 0)),
        out_shape=jax.ShapeDtypeStruct((NL, 1), jnp.int32),
    )(attn)

    pooled = pl.pallas_call(
        _pool_body,
        grid=(C // CB,),
        in_specs=[pl.BlockSpec((CB, H, W), lambda i: (i, 0, 0))],
        out_specs=pl.BlockSpec((CB * HL, WL), lambda i: (i, 0)),
        out_shape=jax.ShapeDtypeStruct((C * HL, WL), jnp.float32),
    )(x)

    gt = _sc_gather(pooled.reshape(C, NL), idx2.reshape(NL))

    out = pl.pallas_call(
        _upsample_body,
        grid=(C // CB,),
        in_specs=[pl.BlockSpec((CB * HL, WL), lambda i: (i, 0))],
        out_specs=pl.BlockSpec((CB, H, W), lambda i: (i, 0, 0)),
        out_shape=jax.ShapeDtypeStruct((C, H, W), jnp.float32),
    )(gt.reshape(C * HL, WL))

    return out.reshape(1, C, H, W)


# speedup vs baseline: 386.7297x; 1.0030x over previous
import functools

import jax
import jax.numpy as jnp
from jax import lax
from jax.experimental import pallas as pl
from jax.experimental.pallas import tpu as pltpu
from jax.experimental.pallas import tpu_sc as plsc

RATIO = 4
C = 96
H = 224
W = 224
HL = H // RATIO
WL = W // RATIO
NL = HL * WL

NC = 2
NS = 16
NW = NC * NS
CPW = C // NW
NVREG = NL // 16

RB = 224
CB = 24

_PREC = lax.Precision.DEFAULT


def _pool_matrices():
    hh = lax.broadcasted_iota(jnp.int32, (H, HL), 0)
    rr = lax.broadcasted_iota(jnp.int32, (H, HL), 1)
    a = (hh // RATIO == rr).astype(jnp.float32)
    tr = lax.broadcasted_iota(jnp.int32, (HL, H), 0)
    th = lax.broadcasted_iota(jnp.int32, (HL, H), 1)
    at = (th // RATIO == tr).astype(jnp.float32)
    return a, at


def _argmax_body(x_ref, o_ref):
    x = x_ref[...]
    mx = jnp.max(x, axis=1, keepdims=True)
    ii = lax.broadcasted_iota(jnp.int32, x.shape, 1)
    cand = jnp.where(x == mx, ii, NL)
    o_ref[...] = jnp.min(cand, axis=1, keepdims=True)


def _pool_body(x_ref, o_ref):
    a, at = _pool_matrices()
    x = x_ref[...]
    u = lax.dot(x.reshape(CB * H, W), a, precision=_PREC)
    zs = [lax.dot(at, u[c * H:(c + 1) * H], precision=_PREC)
          for c in range(CB)]
    z = jnp.concatenate(zs, axis=0)
    o_ref[...] = z * (1.0 / (RATIO * RATIO))


def _upsample_body(g_ref, o_ref):
    a, at = _pool_matrices()
    g = g_ref[...]
    rs = [lax.dot(a, g[c * HL:(c + 1) * HL], precision=_PREC)
          for c in range(CB)]
    r = jnp.concatenate(rs, axis=0)
    z = lax.dot(r, at, precision=_PREC)
    o_ref[...] = z.reshape(CB, H, W)


def _sc_gather_body(pcm_hbm, idx_hbm, out_hbm, tbl_v, idx_v, out_v):
    wid = lax.axis_index("s") * NC + lax.axis_index("c")
    base_c = wid * CPW
    pltpu.sync_copy(pcm_hbm.at[pl.ds(base_c, CPW)], tbl_v)
    pltpu.sync_copy(idx_hbm, idx_v)

    def body(g, carry):
        idx16 = idx_v[pl.ds(g * 16, 16)]
        for c in range(CPW):
            cvec = jnp.full((16,), c, jnp.int32)
            vals = plsc.load_gather(tbl_v, [cvec, idx16])
            out_v[c, pl.ds(g * 16, 16)] = vals
        return carry

    lax.fori_loop(0, NVREG, body, 0, unroll=4)
    pltpu.sync_copy(out_v, out_hbm.at[pl.ds(base_c, CPW)])


def _sc_gather(pcm, idx):
    mesh = plsc.VectorSubcoreMesh(core_axis_name="c", subcore_axis_name="s")
    return pl.kernel(
        _sc_gather_body,
        out_type=jax.ShapeDtypeStruct((C, NL), jnp.float32),
        mesh=mesh,
        scratch_types=[
            pltpu.VMEM((CPW, NL), jnp.float32),
            pltpu.VMEM((NL,), jnp.int32),
            pltpu.VMEM((CPW, NL), jnp.float32),
        ],
        compiler_params=pltpu.CompilerParams(
            use_tc_tiling_on_sc=False, needs_layout_passes=False),
    )(pcm, idx)


def kernel(v_high_feat, coarse_attn_map):
    attn = coarse_attn_map.reshape(NL, NL)
    x = v_high_feat.reshape(C, H, W)

    idx2 = pl.pallas_call(
        _argmax_body,
        grid=(NL // RB,),
        in_specs=[pl.BlockSpec((RB, NL), lambda i: (i, 0))],
        out_specs=pl.BlockSpec((RB, 1), lambda i: (i, 0)),
        out_shape=jax.ShapeDtypeStruct((NL, 1), jnp.int32),
    )(attn)

    pooled = pl.pallas_call(
        _pool_body,
        grid=(C // CB,),
        in_specs=[pl.BlockSpec((CB, H, W), lambda i: (i, 0, 0))],
        out_specs=pl.BlockSpec((CB * HL, WL), lambda i: (i, 0)),
        out_shape=jax.ShapeDtypeStruct((C * HL, WL), jnp.float32),
    )(x)

    gt = _sc_gather(pooled.reshape(C, NL), idx2.reshape(NL))

    out = pl.pallas_call(
        _upsample_body,
        grid=(C // CB,),
        in_specs=[pl.BlockSpec((CB * HL, WL), lambda i: (i, 0))],
        out_specs=pl.BlockSpec((CB, H, W), lambda i: (i, 0, 0)),
        out_shape=jax.ShapeDtypeStruct((C, H, W), jnp.float32),
    )(gt.reshape(C * HL, WL))

    return out.reshape(1, C, H, W)


# speedup vs baseline: 403.7398x; 1.0440x over previous
import functools

import jax
import jax.numpy as jnp
from jax import lax
from jax.experimental import pallas as pl
from jax.experimental.pallas import tpu as pltpu
from jax.experimental.pallas import tpu_sc as plsc

RATIO = 4
C = 96
H = 224
W = 224
HL = H // RATIO
WL = W // RATIO
NL = HL * WL

NC = 2
NS = 16
NW = NC * NS
CPW = C // NW
NVREG = NL // 16

RB = 392
CB = 24

_PREC = lax.Precision.DEFAULT


def _pool_matrices():
    hh = lax.broadcasted_iota(jnp.int32, (H, HL), 0)
    rr = lax.broadcasted_iota(jnp.int32, (H, HL), 1)
    a = (hh // RATIO == rr).astype(jnp.float32)
    tr = lax.broadcasted_iota(jnp.int32, (HL, H), 0)
    th = lax.broadcasted_iota(jnp.int32, (HL, H), 1)
    at = (th // RATIO == tr).astype(jnp.float32)
    return a, at


def _argmax_body(x_ref, o_ref):
    x = x_ref[...]
    mx = jnp.max(x, axis=1, keepdims=True)
    ii = lax.broadcasted_iota(jnp.int32, x.shape, 1)
    cand = jnp.where(x == mx, ii, NL)
    o_ref[...] = jnp.min(cand, axis=1, keepdims=True)


def _pool_body(x_ref, o_ref):
    a, at = _pool_matrices()
    x = x_ref[...]
    u = lax.dot(x.reshape(CB * H, W), a, precision=_PREC)
    zs = [lax.dot(at, u[c * H:(c + 1) * H], precision=_PREC)
          for c in range(CB)]
    z = jnp.concatenate(zs, axis=0)
    o_ref[...] = z * (1.0 / (RATIO * RATIO))


def _upsample_body(g_ref, o_ref):
    a, at = _pool_matrices()
    g = g_ref[...]
    rs = [lax.dot(a, g[c * HL:(c + 1) * HL], precision=_PREC)
          for c in range(CB)]
    r = jnp.concatenate(rs, axis=0)
    z = lax.dot(r, at, precision=_PREC)
    o_ref[...] = z.reshape(CB, H, W)


def _sc_gather_body(pcm_hbm, idx_hbm, out_hbm, tbl_v, idx_v, out_v):
    wid = lax.axis_index("s") * NC + lax.axis_index("c")
    base_c = wid * CPW
    pltpu.sync_copy(pcm_hbm.at[pl.ds(base_c, CPW)], tbl_v)
    pltpu.sync_copy(idx_hbm, idx_v)

    def body(g, carry):
        idx16 = idx_v[pl.ds(g * 16, 16)]
        for c in range(CPW):
            cvec = jnp.full((16,), c, jnp.int32)
            vals = plsc.load_gather(tbl_v, [cvec, idx16])
            out_v[c, pl.ds(g * 16, 16)] = vals
        return carry

    lax.fori_loop(0, NVREG, body, 0, unroll=4)
    pltpu.sync_copy(out_v, out_hbm.at[pl.ds(base_c, CPW)])


def _sc_gather(pcm, idx):
    mesh = plsc.VectorSubcoreMesh(core_axis_name="c", subcore_axis_name="s")
    return pl.kernel(
        _sc_gather_body,
        out_type=jax.ShapeDtypeStruct((C, NL), jnp.float32),
        mesh=mesh,
        scratch_types=[
            pltpu.VMEM((CPW, NL), jnp.float32),
            pltpu.VMEM((NL,), jnp.int32),
            pltpu.VMEM((CPW, NL), jnp.float32),
        ],
        compiler_params=pltpu.CompilerParams(
            use_tc_tiling_on_sc=False, needs_layout_passes=False),
    )(pcm, idx)


def kernel(v_high_feat, coarse_attn_map):
    attn = coarse_attn_map.reshape(NL, NL)
    x = v_high_feat.reshape(C, H, W)

    idx2 = pl.pallas_call(
        _argmax_body,
        grid=(NL // RB,),
        in_specs=[pl.BlockSpec((RB, NL), lambda i: (i, 0))],
        out_specs=pl.BlockSpec((RB, 1), lambda i: (i, 0)),
        out_shape=jax.ShapeDtypeStruct((NL, 1), jnp.int32),
    )(attn)

    pooled = pl.pallas_call(
        _pool_body,
        grid=(C // CB,),
        in_specs=[pl.BlockSpec((CB, H, W), lambda i: (i, 0, 0))],
        out_specs=pl.BlockSpec((CB * HL, WL), lambda i: (i, 0)),
        out_shape=jax.ShapeDtypeStruct((C * HL, WL), jnp.float32),
    )(x)

    gt = _sc_gather(pooled.reshape(C, NL), idx2.reshape(NL))

    out = pl.pallas_call(
        _upsample_body,
        grid=(C // CB,),
        in_specs=[pl.BlockSpec((CB * HL, WL), lambda i: (i, 0))],
        out_specs=pl.BlockSpec((CB, H, W), lambda i: (i, 0, 0)),
        out_shape=jax.ShapeDtypeStruct((C, H, W), jnp.float32),
    )(gt.reshape(C * HL, WL))

    return out.reshape(1, C, H, W)
